# baseline (speedup 1.0000x reference)
import jax
import jax.numpy as jnp
from jax.experimental import pallas as pl
from jax.experimental.pallas import tpu as pltpu

_LANES = 128
_SPLIT = 16
_C = 32
_K = 9


def _softmax_pallas(x, num_chunks, num_cached):
    n = x.shape[0]
    rows = n // _LANES
    r_rows = rows // num_chunks
    x3 = x.reshape(num_chunks, r_rows, _LANES)

    def fused_kernel(x_ref, o_ref, mx_s, sx_s, cache):
        p = pl.program_id(0)
        c = pl.program_id(1)
        cc = jnp.minimum(c, num_cached - 1)

        @pl.when(p == 0)
        def _():
            v = x_ref[0]
            v3 = v.reshape(_SPLIT, v.shape[0] // _SPLIT, _LANES)
            m3 = jnp.max(v3, axis=1)
            m = jnp.max(m3, axis=0, keepdims=True)
            s3 = jnp.sum(jnp.exp(v3 - m[None]), axis=1)
            s = jnp.sum(s3, axis=0, keepdims=True)
            mx_s[pl.ds(c, 1)] = m[None]
            sx_s[pl.ds(c, 1)] = s[None]

            @pl.when(c < num_cached)
            def _():
                cache[pl.ds(cc, 1)] = x_ref[...]

        @pl.when(p == 1)
        def _():
            mp = mx_s[:, 0, :]
            sp = sx_s[:, 0, :]
            m_gl = jnp.max(jnp.max(mp, axis=0, keepdims=True),
                           axis=1, keepdims=True)
            s_gl = jnp.sum(
                jnp.sum(sp * jnp.exp(mp - m_gl), axis=0, keepdims=True),
                axis=1, keepdims=True)
            r = 1.0 / s_gl

            @pl.when(c < num_cached)
            def _():
                o_ref[0] = jnp.exp(cache[cc] - m_gl) * r

            @pl.when(c >= num_cached)
            def _():
                o_ref[0] = jnp.exp(x_ref[0] - m_gl) * r

    out3 = pl.pallas_call(
        fused_kernel,
        out_shape=jax.ShapeDtypeStruct((num_chunks, r_rows, _LANES),
                                       jnp.float32),
        grid=(2, num_chunks),
        in_specs=[pl.BlockSpec(
            (1, r_rows, _LANES),
            lambda p, c: (jnp.where((p == 1) & (c < num_cached),
                                    num_cached, c), 0, 0))],
        out_specs=pl.BlockSpec(
            (1, r_rows, _LANES),
            lambda p, c: (jnp.where(p == 0, 0, c), 0, 0)),
        scratch_shapes=[
            pltpu.VMEM((num_chunks, 1, _LANES), jnp.float32),
            pltpu.VMEM((num_chunks, 1, _LANES), jnp.float32),
            pltpu.VMEM((num_cached, r_rows, _LANES), jnp.float32),
        ],
        compiler_params=pltpu.CompilerParams(
            dimension_semantics=("arbitrary", "arbitrary"),
            vmem_limit_bytes=60000 * 1024),
        name="softmax_fused",
    )(x3)

    return out3.reshape(n)


def kernel(x):
    return _softmax_pallas(x, num_chunks=_C, num_cached=_K)


# speedup vs baseline: 1.0338x; 1.0338x over previous
import jax
import jax.numpy as jnp
from jax.experimental import pallas as pl
from jax.experimental.pallas import tpu as pltpu

_LANES = 128
_SPLIT = 16
_C = 16
_K = 2


def _softmax_pallas(x, num_chunks, num_cached):
    n = x.shape[0]
    rows = n // _LANES
    r_rows = rows // num_chunks
    x3 = x.reshape(num_chunks, r_rows, _LANES)

    def fused_kernel(x_ref, o_ref, mx_s, sx_s, cache):
        p = pl.program_id(0)
        c = pl.program_id(1)
        cc = jnp.minimum(c, num_cached - 1)

        @pl.when(p == 0)
        def _():
            v = x_ref[0]
            v3 = v.reshape(_SPLIT, v.shape[0] // _SPLIT, _LANES)
            m3 = jnp.max(v3, axis=1)
            m = jnp.max(m3, axis=0, keepdims=True)
            s3 = jnp.sum(jnp.exp(v3 - m[None]), axis=1)
            s = jnp.sum(s3, axis=0, keepdims=True)
            mx_s[pl.ds(c, 1)] = m[None]
            sx_s[pl.ds(c, 1)] = s[None]

            @pl.when(c < num_cached)
            def _():
                cache[pl.ds(cc, 1)] = x_ref[...]

        @pl.when(p == 1)
        def _():
            mp = mx_s[:, 0, :]
            sp = sx_s[:, 0, :]
            m_gl = jnp.max(jnp.max(mp, axis=0, keepdims=True),
                           axis=1, keepdims=True)
            s_gl = jnp.sum(
                jnp.sum(sp * jnp.exp(mp - m_gl), axis=0, keepdims=True),
                axis=1, keepdims=True)
            r = 1.0 / s_gl

            @pl.when(c < num_cached)
            def _():
                o_ref[0] = jnp.exp(cache[cc] - m_gl) * r

            @pl.when(c >= num_cached)
            def _():
                o_ref[0] = jnp.exp(x_ref[0] - m_gl) * r

    out3 = pl.pallas_call(
        fused_kernel,
        out_shape=jax.ShapeDtypeStruct((num_chunks, r_rows, _LANES),
                                       jnp.float32),
        grid=(2, num_chunks),
        in_specs=[pl.BlockSpec(
            (1, r_rows, _LANES),
            lambda p, c: (jnp.where((p == 1) & (c < num_cached),
                                    num_cached, c), 0, 0))],
        out_specs=pl.BlockSpec(
            (1, r_rows, _LANES),
            lambda p, c: (jnp.where(p == 0, 0, c), 0, 0)),
        scratch_shapes=[
            pltpu.VMEM((num_chunks, 1, _LANES), jnp.float32),
            pltpu.VMEM((num_chunks, 1, _LANES), jnp.float32),
            pltpu.VMEM((num_cached, r_rows, _LANES), jnp.float32),
        ],
        compiler_params=pltpu.CompilerParams(
            dimension_semantics=("arbitrary", "arbitrary"),
            vmem_limit_bytes=60000 * 1024),
        name="softmax_fused",
    )(x3)

    return out3.reshape(n)


def kernel(x):
    return _softmax_pallas(x, num_chunks=_C, num_cached=_K)


# speedup vs baseline: 1.0518x; 1.0174x over previous
import jax
import jax.numpy as jnp
from jax.experimental import pallas as pl
from jax.experimental.pallas import tpu as pltpu

_LANES = 128
_SPLIT = 16
_C = 16
_K = 3


def _softmax_pallas(x, num_chunks, num_cached):
    n = x.shape[0]
    rows = n // _LANES
    r_rows = rows // num_chunks
    x3 = x.reshape(num_chunks, r_rows, _LANES)

    def fused_kernel(x_ref, o_ref, mx_s, sx_s, cache):
        p = pl.program_id(0)
        c = pl.program_id(1)
        cc = jnp.minimum(c, num_cached - 1)

        @pl.when(p == 0)
        def _():
            v = x_ref[0]
            v3 = v.reshape(_SPLIT, v.shape[0] // _SPLIT, _LANES)
            m3 = jnp.max(v3, axis=1)
            m = jnp.max(m3, axis=0, keepdims=True)
            s3 = jnp.sum(jnp.exp(v3 - m[None]), axis=1)
            s = jnp.sum(s3, axis=0, keepdims=True)
            mx_s[pl.ds(c, 1)] = m[None]
            sx_s[pl.ds(c, 1)] = s[None]

            @pl.when(c < num_cached)
            def _():
                cache[pl.ds(cc, 1)] = x_ref[...]

        @pl.when(p == 1)
        def _():
            mp = mx_s[:, 0, :]
            sp = sx_s[:, 0, :]
            m_gl = jnp.max(jnp.max(mp, axis=0, keepdims=True),
                           axis=1, keepdims=True)
            s_gl = jnp.sum(
                jnp.sum(sp * jnp.exp(mp - m_gl), axis=0, keepdims=True),
                axis=1, keepdims=True)
            r = 1.0 / s_gl

            @pl.when(c < num_cached)
            def _():
                o_ref[0] = jnp.exp(cache[cc] - m_gl) * r

            @pl.when(c >= num_cached)
            def _():
                o_ref[0] = jnp.exp(x_ref[0] - m_gl) * r

    out3 = pl.pallas_call(
        fused_kernel,
        out_shape=jax.ShapeDtypeStruct((num_chunks, r_rows, _LANES),
                                       jnp.float32),
        grid=(2, num_chunks),
        in_specs=[pl.BlockSpec(
            (1, r_rows, _LANES),
            lambda p, c: (jnp.where((p == 1) & (c < num_cached),
                                    num_cached, c), 0, 0))],
        out_specs=pl.BlockSpec(
            (1, r_rows, _LANES),
            lambda p, c: (jnp.where(p == 0, 0, c), 0, 0)),
        scratch_shapes=[
            pltpu.VMEM((num_chunks, 1, _LANES), jnp.float32),
            pltpu.VMEM((num_chunks, 1, _LANES), jnp.float32),
            pltpu.VMEM((num_cached, r_rows, _LANES), jnp.float32),
        ],
        compiler_params=pltpu.CompilerParams(
            dimension_semantics=("arbitrary", "arbitrary"),
            vmem_limit_bytes=64 * 1024 * 1024),
        name="softmax_fused",
    )(x3)

    return out3.reshape(n)


def kernel(x):
    return _softmax_pallas(x, num_chunks=_C, num_cached=_K)


# speedup vs baseline: 1.1215x; 1.0663x over previous
import jax
import jax.numpy as jnp
from jax.experimental import pallas as pl
from jax.experimental.pallas import tpu as pltpu

_LANES = 128
_SPLIT = 16
_C = 16
_K = 6


def _softmax_pallas(x, num_chunks, num_cached):
    n = x.shape[0]
    rows = n // _LANES
    r_rows = rows // num_chunks
    x3 = x.reshape(num_chunks, r_rows, _LANES)

    def fused_kernel(x_ref, o_ref, mx_s, sx_s, cache):
        p = pl.program_id(0)
        c = pl.program_id(1)
        cc = jnp.minimum(c, num_cached - 1)

        @pl.when(p == 0)
        def _():
            v = x_ref[0]
            v3 = v.reshape(_SPLIT, v.shape[0] // _SPLIT, _LANES)
            m3 = jnp.max(v3, axis=1)
            m = jnp.max(m3, axis=0, keepdims=True)
            s3 = jnp.sum(jnp.exp(v3 - m[None]), axis=1)
            s = jnp.sum(s3, axis=0, keepdims=True)
            mx_s[pl.ds(c, 1)] = m[None]
            sx_s[pl.ds(c, 1)] = s[None]

            @pl.when(c < num_cached)
            def _():
                cache[pl.ds(cc, 1)] = x_ref[...].astype(jnp.bfloat16)

        @pl.when(p == 1)
        def _():
            mp = mx_s[:, 0, :]
            sp = sx_s[:, 0, :]
            m_gl = jnp.max(jnp.max(mp, axis=0, keepdims=True),
                           axis=1, keepdims=True)
            s_gl = jnp.sum(
                jnp.sum(sp * jnp.exp(mp - m_gl), axis=0, keepdims=True),
                axis=1, keepdims=True)
            r = 1.0 / s_gl

            @pl.when(c < num_cached)
            def _():
                o_ref[0] = jnp.exp(cache[cc].astype(jnp.float32) - m_gl) * r

            @pl.when(c >= num_cached)
            def _():
                o_ref[0] = jnp.exp(x_ref[0] - m_gl) * r

    out3 = pl.pallas_call(
        fused_kernel,
        out_shape=jax.ShapeDtypeStruct((num_chunks, r_rows, _LANES),
                                       jnp.float32),
        grid=(2, num_chunks),
        in_specs=[pl.BlockSpec(
            (1, r_rows, _LANES),
            lambda p, c: (jnp.where((p == 1) & (c < num_cached),
                                    num_cached, c), 0, 0))],
        out_specs=pl.BlockSpec(
            (1, r_rows, _LANES),
            lambda p, c: (jnp.where(p == 0, 0, c), 0, 0)),
        scratch_shapes=[
            pltpu.VMEM((num_chunks, 1, _LANES), jnp.float32),
            pltpu.VMEM((num_chunks, 1, _LANES), jnp.float32),
            pltpu.VMEM((num_cached, r_rows, _LANES), jnp.bfloat16),
        ],
        compiler_params=pltpu.CompilerParams(
            dimension_semantics=("arbitrary", "arbitrary"),
            vmem_limit_bytes=64 * 1024 * 1024),
        name="softmax_fused",
    )(x3)

    return out3.reshape(n)


def kernel(x):
    return _softmax_pallas(x, num_chunks=_C, num_cached=_K)
